# baseline (speedup 1.0000x reference)
import functools

import jax
import jax.numpy as jnp
from jax import lax
from jax.experimental import pallas as pl
from jax.experimental.pallas import tpu as pltpu
from jax.experimental.pallas import tpu_sc as plsc

NC = 2
NS = 16
NW = NC * NS
LANES = 16


def _ceil16_offsets(ctx):
  offs = []
  k = 0
  while k + LANES <= ctx:
    offs.append(k)
    k += LANES
  if k < ctx:
    offs.append(ctx - LANES)
  return offs


def _sc_body(ctx, dim, n_chunks, text_hbm, tp_hbm, pos_hbm,
             emb_out, len_out, idx_a, idx_b, pair_a, pair_b, par_a, par_b,
             g_a, g_b, st_a, st_b, pos_v, len_v,
             sem_i_a, sem_i_b, sem_g_a, sem_g_b, sem_o_a, sem_o_b, sem_p):
  wid = lax.axis_index("s") * NC + lax.axis_index("c")
  w_row0 = wid * (n_chunks * ctx)
  iota = lax.broadcasted_iota(jnp.int32, (LANES,), 0)
  tail_offs = _ceil16_offsets(ctx)

  pltpu.async_copy(pos_hbm, pos_v, sem_p).wait()

  lo = (ctx // 2 + 7) // 8 * 8
  hi = ctx - lo
  assert lo % 8 == 0 and 0 < hi <= 128 and lo <= 128

  def start_idx(c, ref, sem):
    base = pl.multiple_of(w_row0 + c * ctx, 8)
    pltpu.async_copy(text_hbm.at[pl.ds(base, ctx)], ref, sem)

  def drain_idx(ref, sem):
    pltpu.make_async_copy(text_hbm.at[pl.ds(0, ctx)], ref, sem).wait()

  def make_pairs(idx_ref, pair_ref, par_ref):
    for off in tail_offs:
      ii = idx_ref[pl.ds(off, LANES)]
      pair_ref[pl.ds(off, LANES)] = ii >> 1
      par_ref[pl.ds(off, LANES)] = ii & 1

  def start_gather(pair_ref, g_ref, sem):
    pltpu.async_copy(tp_hbm.at[pair_ref.at[pl.ds(0, lo)]],
                     g_ref.at[pl.ds(0, lo)], sem)
    pltpu.async_copy(tp_hbm.at[pair_ref.at[pl.ds(lo, hi)]],
                     g_ref.at[pl.ds(lo, hi)], sem)

  def drain_gather(g_ref, sem):
    pltpu.make_async_copy(tp_hbm.at[pl.ds(0, ctx)], g_ref, sem).wait()

  def start_out(c, st_ref, sem):
    base = pl.multiple_of(w_row0 + c * ctx, 8)
    pltpu.async_copy(st_ref, emb_out.at[pl.ds(base, ctx)], sem)

  def drain_out(st_ref, sem):
    pltpu.make_async_copy(st_ref, emb_out.at[pl.ds(0, ctx)], sem).wait()

  def reduce16(vec, op):
    m = vec[0]
    for k in range(1, LANES):
      m = op(m, vec[k])
    return m

  def seq_argmax(idx_ref):
    vmax = idx_ref[pl.ds(tail_offs[0], LANES)]
    for off in tail_offs[1:]:
      vmax = jnp.maximum(vmax, idx_ref[pl.ds(off, LANES)])
    m = reduce16(vmax, jnp.maximum)
    vpos = jnp.full((LANES,), jnp.int32(0x7FFFFFFF), dtype=jnp.int32)
    for off in tail_offs:
      vv = idx_ref[pl.ds(off, LANES)]
      vpos = jnp.minimum(vpos, jnp.where(vv == m, iota + off, 0x7FFFFFFF))
    return reduce16(vpos, jnp.minimum)

  def stage_rows(g_ref, st_ref, par_ref):
    def body(k, carry):
      off = jnp.minimum(k * LANES, ctx - LANES)
      parv = par_ref[pl.ds(off, LANES)]
      for j in range(LANES):
        t = off + j
        cbase = parv[j] * dim
        for q in range(dim // LANES):
          sl = pl.ds(q * LANES, LANES)
          vals = g_ref[t, pl.ds(cbase + q * LANES, LANES)]
          st_ref[t, sl] = vals + pos_v[t // 2,
                                       pl.ds((t % 2) * dim + q * LANES,
                                             LANES)]
      return carry
    lax.fori_loop(0, len(tail_offs), body, 0)

  def do_chunk(c, p, acc):
    idx_ref = (idx_a, idx_b)[p]
    pair_ref = (pair_a, pair_b)[p]
    par_ref = (par_a, par_b)[p]
    g_ref = (g_a, g_b)[p]
    st_ref = (st_a, st_b)[p]
    s_i = (sem_i_a, sem_i_b)[p]
    s_g = (sem_g_a, sem_g_b)[p]
    s_o = (sem_o_a, sem_o_b)[p]

    drain_idx(idx_ref, s_i)
    make_pairs(idx_ref, pair_ref, par_ref)
    start_gather(pair_ref, g_ref, s_g)

    @pl.when(c + 1 < n_chunks)
    def _():
      start_idx(c + 1, (idx_a, idx_b)[1 - p], (sem_i_a, sem_i_b)[1 - p])

    r = seq_argmax(idx_ref)
    acc = jnp.where(iota == c % LANES, r, acc)

    @pl.when(c % LANES == LANES - 1)
    def _():
      len_v[pl.ds((c // LANES) * LANES, LANES)] = acc

    drain_gather(g_ref, s_g)

    @pl.when(c >= 2)
    def _():
      drain_out(st_ref, s_o)

    stage_rows(g_ref, st_ref, par_ref)
    start_out(c, st_ref, s_o)
    return acc

  start_idx(0, idx_a, sem_i_a)

  def outer(i, acc):
    acc = do_chunk(2 * i, 0, acc)
    acc = do_chunk(2 * i + 1, 1, acc)
    return acc

  lax.fori_loop(0, n_chunks // 2, outer, jnp.zeros((LANES,), jnp.int32))

  drain_out(st_a, sem_o_a)
  drain_out(st_b, sem_o_b)
  pltpu.sync_copy(len_v, len_out.at[pl.ds(wid * n_chunks, n_chunks)])


def _mask_body(ctx, o_ref):
  r = lax.broadcasted_iota(jnp.int32, (ctx, ctx), 0)
  c = lax.broadcasted_iota(jnp.int32, (ctx, ctx), 1)
  o_ref[...] = jnp.where(c > r, -jnp.inf, 0.0).astype(jnp.float32)


def kernel(text, token_embedding, pos_embed):
  b, ctx = text.shape
  v, dim = token_embedding.shape

  rows_total = b * ctx
  n_chunks = b // NW
  assert b % NW == 0 and n_chunks % (2 * LANES) == 0
  assert dim % LANES == 0 and ctx % 8 == 0 and v % 2 == 0

  text_flat = text.reshape(rows_total).astype(jnp.int32)
  table_pairs = token_embedding.reshape(v // 2, 2 * dim)
  pos128 = pos_embed.reshape(ctx // 2, 2 * dim)

  mesh = plsc.VectorSubcoreMesh(core_axis_name="c", subcore_axis_name="s")
  sc = pl.kernel(
      functools.partial(_sc_body, ctx, dim, n_chunks),
      out_type=(
          jax.ShapeDtypeStruct((rows_total, dim), jnp.float32),
          jax.ShapeDtypeStruct((b,), jnp.int32),
      ),
      mesh=mesh,
      compiler_params=pltpu.CompilerParams(use_tc_tiling_on_sc=True,
                                           needs_layout_passes=False),
      scratch_types=[
          pltpu.VMEM((ctx,), jnp.int32),
          pltpu.VMEM((ctx,), jnp.int32),
          pltpu.VMEM((ctx,), jnp.int32),
          pltpu.VMEM((ctx,), jnp.int32),
          pltpu.VMEM((ctx,), jnp.int32),
          pltpu.VMEM((ctx,), jnp.int32),
          pltpu.VMEM((ctx, 2 * dim), jnp.float32),
          pltpu.VMEM((ctx, 2 * dim), jnp.float32),
          pltpu.VMEM((ctx, dim), jnp.float32),
          pltpu.VMEM((ctx, dim), jnp.float32),
          pltpu.VMEM((ctx // 2, 2 * dim), jnp.float32),
          pltpu.VMEM((b // NW,), jnp.int32),
          pltpu.SemaphoreType.DMA,
          pltpu.SemaphoreType.DMA,
          pltpu.SemaphoreType.DMA,
          pltpu.SemaphoreType.DMA,
          pltpu.SemaphoreType.DMA,
          pltpu.SemaphoreType.DMA,
          pltpu.SemaphoreType.DMA,
      ],
  )
  emb_flat, lengths = sc(text_flat, table_pairs, pos128)
  token_text = emb_flat.reshape(b, ctx, dim)

  mask = pl.pallas_call(
      functools.partial(_mask_body, ctx),
      out_shape=jax.ShapeDtypeStruct((ctx, ctx), jnp.float32),
  )()

  return token_text, lengths, mask
